# baseline (speedup 1.0000x reference)
import functools

import jax
import jax.numpy as jnp
from jax import lax
from jax.experimental import pallas as pl
from jax.experimental.pallas import tpu as pltpu, tpu_sc as plsc

N = 10000
E = 320000
D = 128
G = 64

F32 = jnp.float32

_NC = 2
_NS = 16
_NW = _NC * _NS
_NPAD = 10240
_RPW = _NPAD // _NW
_EPW = E // _NW
_CH = 80
_NCHUNK = _EPW // _CH
_SCH = 512
_NSCAN = E // _SCH

_iota16 = functools.partial(lax.iota, jnp.int32)


def _splat(x):
    return lax.broadcast(x, (16,))


def _gather_scale_kernel(x_hbm, src_hbm, ea_hbm, z_hbm, src_v, ea_v, rows_v,
                         sem):
    c = lax.axis_index("c")
    s = lax.axis_index("s")
    wid = c * _NS + s

    def chunk(k, carry):
        base = wid * _EPW + k * _CH
        pltpu.sync_copy(src_hbm.at[pl.ds(base, _CH)], src_v)
        pltpu.sync_copy(ea_hbm.at[pl.ds(base, _CH)], ea_v)
        pltpu.async_copy(x_hbm.at[src_v], rows_v, sem).wait()

        def edge(i, carry2):
            ri = _splat(i)
            eab = plsc.load_gather(ea_v, [ri])
            for j in range(8):
                ci = j * 16 + _iota16(16)
                v = plsc.load_gather(rows_v, [ri, ci])
                plsc.store_scatter(rows_v, [ri, ci], v * eab)
            return carry2

        lax.fori_loop(0, _CH, edge, 0)
        pltpu.sync_copy(rows_v, z_hbm.at[pl.ds(base, _CH)])
        return carry

    lax.fori_loop(0, _NCHUNK, chunk, 0)


@jax.jit
def _gather_scale(x, src, ea):
    mesh = plsc.VectorSubcoreMesh(core_axis_name="c", subcore_axis_name="s")
    f = functools.partial(
        pl.kernel,
        mesh=mesh,
        out_type=jax.ShapeDtypeStruct((E, D), F32),
        scratch_types=[
            pltpu.VMEM((_CH,), jnp.int32),
            pltpu.VMEM((_CH,), F32),
            pltpu.VMEM((_CH, D), F32),
            pltpu.SemaphoreType.DMA,
        ],
        compiler_params=pltpu.CompilerParams(needs_layout_passes=False),
    )(_gather_scale_kernel)
    return f(x, src, ea)


def _aggregate_kernel(m_hbm, dst_hbm, zero_hbm, a_hbm,
                      dstc_v, elist_v, dloc_v, rows_v, acc_v, sem):
    c = lax.axis_index("c")
    s = lax.axis_index("s")
    wid = c * _NS + s
    lo = wid * _RPW

    pltpu.sync_copy(zero_hbm.at[pl.ds(0, _RPW)], acc_v)
    zero16i = jnp.zeros((16,), jnp.int32)
    for b in range(_SCH // 16):
        elist_v[pl.ds(b * 16, 16)] = zero16i
        dloc_v[pl.ds(b * 16, 16)] = zero16i

    los = _splat(lo)
    his = _splat(lo + _RPW)

    def chunk(kc, carry):
        base = kc * _SCH
        pltpu.sync_copy(dst_hbm.at[pl.ds(base, _SCH)], dstc_v)
        def scan16(b, kpos):
            v = dstc_v[pl.ds(b * 16, 16)]
            mask = (v >= los) & (v < his)
            pref = plsc.cumsum(jnp.where(mask, 1, 0))
            pos = _splat(kpos) + pref - 1
            evals = _splat(base + b * 16) + _iota16(16)
            plsc.store_scatter(elist_v, [pos], evals, mask=mask)
            plsc.store_scatter(dloc_v, [pos], v - los, mask=mask)
            return kpos + jnp.max(pref)

        k = lax.fori_loop(0, _SCH // 16, scan16, jnp.int32(0))

        def sub(jg, carry2):
            cp = pltpu.async_copy(
                m_hbm.at[elist_v.at[pl.ds(jg * 16, 16)]],
                rows_v.at[pl.ds(jg * 16, 16)], sem)
            cp.wait()
            return carry2

        lax.fori_loop(0, (k + 15) // 16, sub, 0)

        def edge(i, carry2):
            ri = _splat(i)
            dl = plsc.load_gather(dloc_v, [ri])
            for j in range(8):
                ci = j * 16 + _iota16(16)
                v = plsc.load_gather(rows_v, [ri, ci])
                plsc.addupdate_scatter(acc_v, [dl, ci], v)
            return carry2

        lax.fori_loop(0, k, edge, 0)
        return carry

    lax.fori_loop(0, _NSCAN, chunk, 0)
    pltpu.sync_copy(acc_v, a_hbm.at[pl.ds(lo, _RPW)])


@jax.jit
def _aggregate(m, dst, zero):
    mesh = plsc.VectorSubcoreMesh(core_axis_name="c", subcore_axis_name="s")
    f = functools.partial(
        pl.kernel,
        mesh=mesh,
        out_type=jax.ShapeDtypeStruct((_NPAD, D), F32),
        scratch_types=[
            pltpu.VMEM((_SCH,), jnp.int32),
            pltpu.VMEM((_SCH,), jnp.int32),
            pltpu.VMEM((_SCH,), jnp.int32),
            pltpu.VMEM((_SCH, D), F32),
            pltpu.VMEM((_RPW, D), F32),
            pltpu.SemaphoreType.DMA,
        ],
        compiler_params=pltpu.CompilerParams(needs_layout_passes=False),
    )(_aggregate_kernel)
    return f(m, dst, zero)


_BME = 2000
_BM = 2000


def _dot(a, b):
    return jnp.dot(a, b, preferred_element_type=F32)


def _edge_mlp_body(z_ref, we1_ref, be1_ref, we2_ref, be2_ref, m_ref):
    h = jnp.maximum(_dot(z_ref[...], we1_ref[...]) + be1_ref[...], 0.0)
    m_ref[...] = _dot(h, we2_ref[...]) + be2_ref[...]


def _edge_mlp(z, we1, be1, we2, be2):
    vec = pl.BlockSpec((1, D), lambda i: (0, 0))
    mat = pl.BlockSpec((D, D), lambda i: (0, 0))
    blk = pl.BlockSpec((_BME, D), lambda i: (i, 0))
    return pl.pallas_call(
        _edge_mlp_body,
        grid=(E // _BME,),
        in_specs=[blk, mat, vec, mat, vec],
        out_specs=blk,
        out_shape=jax.ShapeDtypeStruct((E, D), F32),
    )(z, we1, be1, we2, be2)


def _tmat_body(x_ref, a_ref, wn1_ref, bn1_ref, t_ref):
    h = x_ref[...] + a_ref[...]
    t_ref[...] = _dot(h, wn1_ref[...]) + bn1_ref[...]


def _tmat(x, a, wn1, bn1):
    vec = pl.BlockSpec((1, D), lambda i: (0, 0))
    mat = pl.BlockSpec((D, D), lambda i: (0, 0))
    blk = pl.BlockSpec((_BM, D), lambda i: (i, 0))
    return pl.pallas_call(
        _tmat_body,
        grid=(N // _BM,),
        in_specs=[blk, blk, mat, vec],
        out_specs=blk,
        out_shape=jax.ShapeDtypeStruct((N, D), F32),
    )(x, a, wn1, bn1)


def _apply_body(t_ref, mu_ref, var_ref, g_ref, bt_ref, wn2_ref, bn2_ref,
                x1_ref):
    th = ((t_ref[...] - mu_ref[...]) * lax.rsqrt(var_ref[...] + 1e-5)
          * g_ref[...] + bt_ref[...])
    u = _dot(jnp.maximum(th, 0.0), wn2_ref[...]) + bn2_ref[...]
    x1_ref[...] = jnp.maximum(u, 0.0)


def _apply(t, mu, var, g, bt, wn2, bn2):
    vec = pl.BlockSpec((1, D), lambda i: (0, 0))
    mat = pl.BlockSpec((D, D), lambda i: (0, 0))
    blk = pl.BlockSpec((_BM, D), lambda i: (i, 0))
    return pl.pallas_call(
        _apply_body,
        grid=(N // _BM,),
        in_specs=[blk, vec, vec, vec, vec, mat, vec],
        out_specs=blk,
        out_shape=jax.ShapeDtypeStruct((N, D), F32),
    )(t, mu, var, g, bt, wn2, bn2)


def _pool_body(hf_ref, p_ref, psum_ref, pcnt_ref):
    hf = hf_ref[...]
    p = p_ref[...]

    @pl.when(pl.program_id(0) == 0)
    def _():
        psum_ref[...] = jnp.zeros_like(psum_ref)
        pcnt_ref[...] = jnp.zeros_like(pcnt_ref)

    psum_ref[...] += lax.dot_general(p, hf, (((0,), (0,)), ((), ())),
                                     preferred_element_type=F32,
                                     precision=lax.Precision.HIGHEST)
    pcnt_ref[...] += lax.dot_general(p, jnp.ones_like(hf),
                                     (((0,), (0,)), ((), ())),
                                     preferred_element_type=F32)


def _pool(hf, p):
    return pl.pallas_call(
        _pool_body,
        grid=(N // _BM,),
        in_specs=[pl.BlockSpec((_BM, D), lambda i: (i, 0)),
                  pl.BlockSpec((_BM, G), lambda i: (i, 0))],
        out_specs=[pl.BlockSpec((G, D), lambda i: (0, 0)),
                   pl.BlockSpec((G, D), lambda i: (0, 0))],
        out_shape=[jax.ShapeDtypeStruct((G, D), F32),
                   jax.ShapeDtypeStruct((G, D), F32)],
    )(hf, p)


def _final_body(psum_ref, pcnt_ref, wf1_ref, bf1_ref, gf_ref, btf_ref,
                wf2_ref, bf2_ref, o_ref):
    pooled = psum_ref[...] / jnp.maximum(pcnt_ref[...], 1.0)
    o1 = jnp.dot(pooled.astype(jnp.bfloat16),
                 wf1_ref[...].astype(jnp.bfloat16),
                 preferred_element_type=F32) + bf1_ref[...]
    mu = jnp.mean(o1, axis=0, keepdims=True)
    dev = o1 - mu
    var = jnp.mean(dev * dev, axis=0, keepdims=True)
    th = dev * lax.rsqrt(var + 1e-5) * gf_ref[...] + btf_ref[...]
    o_ref[...] = jnp.dot(jnp.maximum(th, 0.0).astype(jnp.bfloat16),
                         wf2_ref[...].astype(jnp.bfloat16),
                         preferred_element_type=F32) + bf2_ref[...]


def _final(psum, pcnt, wf1, bf1, gf, btf, wf2, bf2):
    whole = lambda shape: pl.BlockSpec(shape, lambda: (0,) * len(shape))
    return pl.pallas_call(
        _final_body,
        in_specs=[whole((G, D)), whole((G, D)), whole((D, D)), whole((1, D)),
                  whole((1, D)), whole((1, D)), whole((D, D)), whole((1, D))],
        out_specs=whole((G, D)),
        out_shape=jax.ShapeDtypeStruct((G, D), F32),
    )(psum, pcnt, wf1, bf1, gf, btf, wf2, bf2)


def kernel(x, edge_index, edge_attr, batch,
           We1_0, be1_0, We2_0, be2_0, Wn1_0, bn1_0, g_0, bt_0, Wn2_0, bn2_0,
           We1_1, be1_1, We2_1, be2_1, Wn1_1, bn1_1, g_1, bt_1, Wn2_1, bn2_1,
           Wf1, bf1, gf, btf, Wf2, bf2):
    src = edge_index[0]
    dst = edge_index[1]
    zero = jnp.zeros((_NPAD, D), F32)
    p_onehot = (batch[:, None] == jnp.arange(G, dtype=jnp.int32)[None, :])
    p_onehot = p_onehot.astype(F32)
    row = lambda v: v.reshape(1, -1)

    h = x
    for (we1, be1, we2, be2, wn1, bn1_, g_, bt_, wn2, bn2_) in (
        (We1_0, be1_0, We2_0, be2_0, Wn1_0, bn1_0, g_0, bt_0, Wn2_0, bn2_0),
        (We1_1, be1_1, We2_1, be2_1, Wn1_1, bn1_1, g_1, bt_1, Wn2_1, bn2_1)):
        z = _gather_scale(h, src, edge_attr)
        m = _edge_mlp(z, we1, row(be1), we2, row(be2))
        a = _aggregate(m, dst, zero)[:N]
        t = _tmat(h, a, wn1, row(bn1_))
        mu = jnp.mean(t, axis=0, keepdims=True)
        var = jnp.var(t, axis=0, keepdims=True)
        h = _apply(t, mu, var, row(g_), row(bt_), wn2, row(bn2_))

    psum, pcnt = _pool(h, p_onehot)
    return _final(psum, pcnt, Wf1, row(bf1), row(gf), row(btf), Wf2, row(bf2))


# speedup vs baseline: 1.0123x; 1.0123x over previous
import functools

import jax
import jax.numpy as jnp
from jax import lax
from jax.experimental import pallas as pl
from jax.experimental.pallas import tpu as pltpu, tpu_sc as plsc

N = 10000
E = 320000
D = 128
G = 64

F32 = jnp.float32

_NC = 2
_NS = 16
_NW = _NC * _NS
_NPAD = 10240
_RPW = _NPAD // _NW
_EPW = E // _NW
_CH = 80
_NCHUNK = _EPW // _CH
_SCH = 512
_NSCAN = E // _SCH

_iota16 = functools.partial(lax.iota, jnp.int32)


def _splat(x):
    return lax.broadcast(x, (16,))


def _gather_scale_kernel(x_hbm, src_hbm, ea_hbm, z_hbm, src_v, ea_v, rows_v,
                         sem):
    c = lax.axis_index("c")
    s = lax.axis_index("s")
    wid = c * _NS + s

    def chunk(k, carry):
        base = wid * _EPW + k * _CH
        pltpu.sync_copy(src_hbm.at[pl.ds(base, _CH)], src_v)
        pltpu.sync_copy(ea_hbm.at[pl.ds(base, _CH)], ea_v)
        pltpu.async_copy(x_hbm.at[src_v], rows_v, sem).wait()

        def edge(i, carry2):
            ri = _splat(i)
            eab = plsc.load_gather(ea_v, [ri])
            for j in range(8):
                ci = j * 16 + _iota16(16)
                v = plsc.load_gather(rows_v, [ri, ci])
                plsc.store_scatter(rows_v, [ri, ci], v * eab)
            return carry2

        lax.fori_loop(0, _CH, edge, 0, unroll=4)
        pltpu.sync_copy(rows_v, z_hbm.at[pl.ds(base, _CH)])
        return carry

    lax.fori_loop(0, _NCHUNK, chunk, 0)


@jax.jit
def _gather_scale(x, src, ea):
    mesh = plsc.VectorSubcoreMesh(core_axis_name="c", subcore_axis_name="s")
    f = functools.partial(
        pl.kernel,
        mesh=mesh,
        out_type=jax.ShapeDtypeStruct((E, D), F32),
        scratch_types=[
            pltpu.VMEM((_CH,), jnp.int32),
            pltpu.VMEM((_CH,), F32),
            pltpu.VMEM((_CH, D), F32),
            pltpu.SemaphoreType.DMA,
        ],
        compiler_params=pltpu.CompilerParams(needs_layout_passes=False),
    )(_gather_scale_kernel)
    return f(x, src, ea)


def _aggregate_kernel(m_hbm, dst_hbm, zero_hbm, a_hbm,
                      dstc_v, elist_v, dloc_v, rows_v, acc_v, sem):
    c = lax.axis_index("c")
    s = lax.axis_index("s")
    wid = c * _NS + s
    lo = wid * _RPW

    pltpu.sync_copy(zero_hbm.at[pl.ds(0, _RPW)], acc_v)
    zero16i = jnp.zeros((16,), jnp.int32)
    for b in range(_SCH // 16):
        elist_v[pl.ds(b * 16, 16)] = zero16i
        dloc_v[pl.ds(b * 16, 16)] = zero16i

    los = _splat(lo)
    his = _splat(lo + _RPW)

    def chunk(kc, carry):
        base = kc * _SCH
        pltpu.sync_copy(dst_hbm.at[pl.ds(base, _SCH)], dstc_v)
        def scan16(b, kpos):
            v = dstc_v[pl.ds(b * 16, 16)]
            mask = (v >= los) & (v < his)
            pref = plsc.cumsum(jnp.where(mask, 1, 0))
            pos = _splat(kpos) + pref - 1
            evals = _splat(base + b * 16) + _iota16(16)
            plsc.store_scatter(elist_v, [pos], evals, mask=mask)
            plsc.store_scatter(dloc_v, [pos], v - los, mask=mask)
            return kpos + jnp.max(pref)

        k = lax.fori_loop(0, _SCH // 16, scan16, jnp.int32(0))

        def sub(jg, carry2):
            cp = pltpu.async_copy(
                m_hbm.at[elist_v.at[pl.ds(jg * 16, 16)]],
                rows_v.at[pl.ds(jg * 16, 16)], sem)
            cp.wait()
            return carry2

        lax.fori_loop(0, (k + 15) // 16, sub, 0)

        def edge(i, carry2):
            ri = _splat(i)
            dl = plsc.load_gather(dloc_v, [ri])
            for j in range(8):
                ci = j * 16 + _iota16(16)
                v = plsc.load_gather(rows_v, [ri, ci])
                plsc.addupdate_scatter(acc_v, [dl, ci], v)
            return carry2

        lax.fori_loop(0, k, edge, 0)
        return carry

    lax.fori_loop(0, _NSCAN, chunk, 0)
    pltpu.sync_copy(acc_v, a_hbm.at[pl.ds(lo, _RPW)])


@jax.jit
def _aggregate(m, dst, zero):
    mesh = plsc.VectorSubcoreMesh(core_axis_name="c", subcore_axis_name="s")
    f = functools.partial(
        pl.kernel,
        mesh=mesh,
        out_type=jax.ShapeDtypeStruct((_NPAD, D), F32),
        scratch_types=[
            pltpu.VMEM((_SCH,), jnp.int32),
            pltpu.VMEM((_SCH,), jnp.int32),
            pltpu.VMEM((_SCH,), jnp.int32),
            pltpu.VMEM((_SCH, D), F32),
            pltpu.VMEM((_RPW, D), F32),
            pltpu.SemaphoreType.DMA,
        ],
        compiler_params=pltpu.CompilerParams(needs_layout_passes=False),
    )(_aggregate_kernel)
    return f(m, dst, zero)


_BME = 2000
_BM = 2000


def _dot(a, b):
    return jnp.dot(a, b, preferred_element_type=F32)


def _edge_mlp_body(z_ref, we1_ref, be1_ref, we2_ref, be2_ref, m_ref):
    h = jnp.maximum(_dot(z_ref[...], we1_ref[...]) + be1_ref[...], 0.0)
    m_ref[...] = _dot(h, we2_ref[...]) + be2_ref[...]


def _edge_mlp(z, we1, be1, we2, be2):
    vec = pl.BlockSpec((1, D), lambda i: (0, 0))
    mat = pl.BlockSpec((D, D), lambda i: (0, 0))
    blk = pl.BlockSpec((_BME, D), lambda i: (i, 0))
    return pl.pallas_call(
        _edge_mlp_body,
        grid=(E // _BME,),
        in_specs=[blk, mat, vec, mat, vec],
        out_specs=blk,
        out_shape=jax.ShapeDtypeStruct((E, D), F32),
    )(z, we1, be1, we2, be2)


def _tmat_body(x_ref, a_ref, wn1_ref, bn1_ref, t_ref):
    h = x_ref[...] + a_ref[...]
    t_ref[...] = _dot(h, wn1_ref[...]) + bn1_ref[...]


def _tmat(x, a, wn1, bn1):
    vec = pl.BlockSpec((1, D), lambda i: (0, 0))
    mat = pl.BlockSpec((D, D), lambda i: (0, 0))
    blk = pl.BlockSpec((_BM, D), lambda i: (i, 0))
    return pl.pallas_call(
        _tmat_body,
        grid=(N // _BM,),
        in_specs=[blk, blk, mat, vec],
        out_specs=blk,
        out_shape=jax.ShapeDtypeStruct((N, D), F32),
    )(x, a, wn1, bn1)


def _apply_body(t_ref, mu_ref, var_ref, g_ref, bt_ref, wn2_ref, bn2_ref,
                x1_ref):
    th = ((t_ref[...] - mu_ref[...]) * lax.rsqrt(var_ref[...] + 1e-5)
          * g_ref[...] + bt_ref[...])
    u = _dot(jnp.maximum(th, 0.0), wn2_ref[...]) + bn2_ref[...]
    x1_ref[...] = jnp.maximum(u, 0.0)


def _apply(t, mu, var, g, bt, wn2, bn2):
    vec = pl.BlockSpec((1, D), lambda i: (0, 0))
    mat = pl.BlockSpec((D, D), lambda i: (0, 0))
    blk = pl.BlockSpec((_BM, D), lambda i: (i, 0))
    return pl.pallas_call(
        _apply_body,
        grid=(N // _BM,),
        in_specs=[blk, vec, vec, vec, vec, mat, vec],
        out_specs=blk,
        out_shape=jax.ShapeDtypeStruct((N, D), F32),
    )(t, mu, var, g, bt, wn2, bn2)


def _pool_body(hf_ref, p_ref, psum_ref, pcnt_ref):
    hf = hf_ref[...]
    p = p_ref[...]

    @pl.when(pl.program_id(0) == 0)
    def _():
        psum_ref[...] = jnp.zeros_like(psum_ref)
        pcnt_ref[...] = jnp.zeros_like(pcnt_ref)

    psum_ref[...] += lax.dot_general(p, hf, (((0,), (0,)), ((), ())),
                                     preferred_element_type=F32,
                                     precision=lax.Precision.HIGHEST)
    pcnt_ref[...] += lax.dot_general(p, jnp.ones_like(hf),
                                     (((0,), (0,)), ((), ())),
                                     preferred_element_type=F32)


def _pool(hf, p):
    return pl.pallas_call(
        _pool_body,
        grid=(N // _BM,),
        in_specs=[pl.BlockSpec((_BM, D), lambda i: (i, 0)),
                  pl.BlockSpec((_BM, G), lambda i: (i, 0))],
        out_specs=[pl.BlockSpec((G, D), lambda i: (0, 0)),
                   pl.BlockSpec((G, D), lambda i: (0, 0))],
        out_shape=[jax.ShapeDtypeStruct((G, D), F32),
                   jax.ShapeDtypeStruct((G, D), F32)],
    )(hf, p)


def _final_body(psum_ref, pcnt_ref, wf1_ref, bf1_ref, gf_ref, btf_ref,
                wf2_ref, bf2_ref, o_ref):
    pooled = psum_ref[...] / jnp.maximum(pcnt_ref[...], 1.0)
    o1 = jnp.dot(pooled.astype(jnp.bfloat16),
                 wf1_ref[...].astype(jnp.bfloat16),
                 preferred_element_type=F32) + bf1_ref[...]
    mu = jnp.mean(o1, axis=0, keepdims=True)
    dev = o1 - mu
    var = jnp.mean(dev * dev, axis=0, keepdims=True)
    th = dev * lax.rsqrt(var + 1e-5) * gf_ref[...] + btf_ref[...]
    o_ref[...] = jnp.dot(jnp.maximum(th, 0.0).astype(jnp.bfloat16),
                         wf2_ref[...].astype(jnp.bfloat16),
                         preferred_element_type=F32) + bf2_ref[...]


def _final(psum, pcnt, wf1, bf1, gf, btf, wf2, bf2):
    whole = lambda shape: pl.BlockSpec(shape, lambda: (0,) * len(shape))
    return pl.pallas_call(
        _final_body,
        in_specs=[whole((G, D)), whole((G, D)), whole((D, D)), whole((1, D)),
                  whole((1, D)), whole((1, D)), whole((D, D)), whole((1, D))],
        out_specs=whole((G, D)),
        out_shape=jax.ShapeDtypeStruct((G, D), F32),
    )(psum, pcnt, wf1, bf1, gf, btf, wf2, bf2)


def kernel(x, edge_index, edge_attr, batch,
           We1_0, be1_0, We2_0, be2_0, Wn1_0, bn1_0, g_0, bt_0, Wn2_0, bn2_0,
           We1_1, be1_1, We2_1, be2_1, Wn1_1, bn1_1, g_1, bt_1, Wn2_1, bn2_1,
           Wf1, bf1, gf, btf, Wf2, bf2):
    src = edge_index[0]
    dst = edge_index[1]
    zero = jnp.zeros((_NPAD, D), F32)
    p_onehot = (batch[:, None] == jnp.arange(G, dtype=jnp.int32)[None, :])
    p_onehot = p_onehot.astype(F32)
    row = lambda v: v.reshape(1, -1)

    h = x
    for (we1, be1, we2, be2, wn1, bn1_, g_, bt_, wn2, bn2_) in (
        (We1_0, be1_0, We2_0, be2_0, Wn1_0, bn1_0, g_0, bt_0, Wn2_0, bn2_0),
        (We1_1, be1_1, We2_1, be2_1, Wn1_1, bn1_1, g_1, bt_1, Wn2_1, bn2_1)):
        z = _gather_scale(h, src, edge_attr)
        m = _edge_mlp(z, we1, row(be1), we2, row(be2))
        a = _aggregate(m, dst, zero)[:N]
        t = _tmat(h, a, wn1, row(bn1_))
        mu = jnp.mean(t, axis=0, keepdims=True)
        var = jnp.var(t, axis=0, keepdims=True)
        h = _apply(t, mu, var, row(g_), row(bt_), wn2, row(bn2_))

    psum, pcnt = _pool(h, p_onehot)
    return _final(psum, pcnt, Wf1, row(bf1), row(gf), row(btf), Wf2, row(bf2))
